# speedup vs baseline: 2.2702x; 2.2702x over previous
import functools

import jax
import jax.numpy as jnp
from jax import lax
from jax.experimental import pallas as pl
from jax.experimental.pallas import tpu as pltpu
from jax.experimental.pallas import tpu_sc as plsc

N_NODES = 100000
MEM_DIM = 128
MSG_DIM = 256
B = 16384

NC = 2
NS = 16
NW = NC * NS
B_PER_W = B // NW

_MESH = functools.partial(
    plsc.VectorSubcoreMesh, core_axis_name="c", subcore_axis_name="s"
)


def _worker_id():
  return lax.axis_index("s") * NC + lax.axis_index("c")


@functools.partial(
    pl.kernel,
    mesh=_MESH(),
    out_type=jax.ShapeDtypeStruct((B, MEM_DIM), jnp.float32),
    scratch_types=[
        pltpu.VMEM((B_PER_W,), jnp.int32),
        pltpu.VMEM((B_PER_W, MEM_DIM), jnp.float32),
        pltpu.SemaphoreType.DMA,
    ],
)
def _sc_gather(mem_hbm, nids_hbm, out_hbm, idx_v, rows_v, sem):
  base = _worker_id() * B_PER_W
  pltpu.sync_copy(nids_hbm.at[pl.ds(base, B_PER_W)], idx_v)
  pltpu.async_copy(mem_hbm.at[idx_v], rows_v, sem).wait()
  pltpu.sync_copy(rows_v, out_hbm.at[pl.ds(base, B_PER_W)])


_BM = 1024


def _gru_body(msg_ref, h_ref, wi_ref, wh_ref, bi_ref, bh_ref, out_ref):
  gi = (
      jnp.dot(msg_ref[...], wi_ref[...], preferred_element_type=jnp.float32)
      + bi_ref[...]
  )
  gh = (
      jnp.dot(h_ref[...], wh_ref[...], preferred_element_type=jnp.float32)
      + bh_ref[...]
  )
  H = MEM_DIM
  r = jax.nn.sigmoid(gi[:, :H] + gh[:, :H])
  z = jax.nn.sigmoid(gi[:, H : 2 * H] + gh[:, H : 2 * H])
  n = jnp.tanh(gi[:, 2 * H :] + r * gh[:, 2 * H :])
  out_ref[...] = (1.0 - z) * n + z * h_ref[...]


def _tc_gru(msg, h, wi_t, wh_t, bi, bh):
  grid = (B // _BM,)
  return pl.pallas_call(
      _gru_body,
      grid=grid,
      in_specs=[
          pl.BlockSpec((_BM, MSG_DIM), lambda i: (i, 0)),
          pl.BlockSpec((_BM, MEM_DIM), lambda i: (i, 0)),
          pl.BlockSpec((MSG_DIM, 3 * MEM_DIM), lambda i: (0, 0)),
          pl.BlockSpec((MEM_DIM, 3 * MEM_DIM), lambda i: (0, 0)),
          pl.BlockSpec((1, 3 * MEM_DIM), lambda i: (0, 0)),
          pl.BlockSpec((1, 3 * MEM_DIM), lambda i: (0, 0)),
      ],
      out_specs=pl.BlockSpec((_BM, MEM_DIM), lambda i: (i, 0)),
      out_shape=jax.ShapeDtypeStruct((B, MEM_DIM), jnp.float32),
  )(msg, h, wi_t, wh_t, bi, bh)


@functools.partial(
    pl.kernel,
    mesh=_MESH(),
    out_type=(),
    scratch_types=[
        pltpu.VMEM((B_PER_W,), jnp.int32),
        pltpu.VMEM((B_PER_W, MEM_DIM), jnp.float32),
        pltpu.VMEM((B_PER_W,), jnp.float32),
        pltpu.SemaphoreType.DMA,
    ],
)
def _sc_scatter(nids_hbm, hnew_hbm, tvals_hbm, mem_hbm, lu_hbm,
                idx_v, rows_v, tv_v, sem):
  base = _worker_id() * B_PER_W
  pltpu.sync_copy(nids_hbm.at[pl.ds(base, B_PER_W)], idx_v)
  pltpu.sync_copy(hnew_hbm.at[pl.ds(base, B_PER_W)], rows_v)
  pltpu.async_copy(rows_v, mem_hbm.at[idx_v], sem).wait()
  pltpu.sync_copy(tvals_hbm, tv_v)
  pltpu.async_copy(tv_v, lu_hbm.at[idx_v], sem).wait()


def kernel(unique_nids, unique_msg, time, memory, last_update,
           W_ih, W_hh, b_ih, b_hh):
  nids = unique_nids.astype(jnp.int32)
  h = _sc_gather(memory, nids)
  h_new = _tc_gru(
      unique_msg, h, W_ih.T, W_hh.T,
      b_ih.reshape(1, -1), b_hh.reshape(1, -1),
  )
  tvals = jnp.full((B_PER_W,), time, dtype=jnp.float32)
  mem_ref = jax.new_ref(memory)
  lu_ref = jax.new_ref(last_update)
  _sc_scatter(nids, h_new, tvals, mem_ref, lu_ref)
  return mem_ref[...], lu_ref[...]


# speedup vs baseline: 2.2818x; 1.0051x over previous
import functools

import jax
import jax.numpy as jnp
from jax import lax
from jax.experimental import pallas as pl
from jax.experimental.pallas import tpu as pltpu
from jax.experimental.pallas import tpu_sc as plsc

N_NODES = 100000
MEM_DIM = 128
MSG_DIM = 256
B = 16384

NC = 2
NS = 16
NW = NC * NS
B_PER_W = B // NW
NCHUNK = 4
CHUNK = B_PER_W // NCHUNK

_MESH = functools.partial(
    plsc.VectorSubcoreMesh, core_axis_name="c", subcore_axis_name="s"
)


def _worker_id():
  return lax.axis_index("s") * NC + lax.axis_index("c")


@functools.partial(
    pl.kernel,
    mesh=_MESH(),
    out_type=jax.ShapeDtypeStruct((B, MEM_DIM), jnp.float32),
    scratch_types=[
        pltpu.VMEM((NCHUNK, CHUNK), jnp.int32),
        pltpu.VMEM((B_PER_W, MEM_DIM), jnp.float32),
        pltpu.VMEM((CHUNK,), jnp.float32),
    ]
    + [pltpu.SemaphoreType.DMA] * 6,
)
def _sc_gather(mem_hbm, nids_hbm, tvals_hbm, lu_hbm, out_hbm,
               idx_v, rows_v, tv_v, s0, s1, s2, s3, ss, sl):
  wid = _worker_id()
  base = wid * B_PER_W
  pltpu.sync_copy(nids_hbm.at[wid], idx_v)
  pltpu.sync_copy(tvals_hbm, tv_v)
  sems = (s0, s1, s2, s3)
  gathers = []
  for k in range(NCHUNK):
    gathers.append(
        pltpu.async_copy(
            mem_hbm.at[idx_v.at[k]],
            rows_v.at[pl.ds(k * CHUNK, CHUNK)],
            sems[k],
        )
    )
  lu_writes = [
      pltpu.async_copy(tv_v, lu_hbm.at[idx_v.at[k]], sl)
      for k in range(NCHUNK)
  ]
  stores = []
  for k in range(NCHUNK):
    gathers[k].wait()
    stores.append(
        pltpu.async_copy(
            rows_v.at[pl.ds(k * CHUNK, CHUNK)],
            out_hbm.at[pl.ds(base + k * CHUNK, CHUNK)],
            ss,
        )
    )
  for c in stores:
    c.wait()
  for c in lu_writes:
    c.wait()


_BM = 1024


def _gru_body(msg_ref, h_ref, wi_ref, wh_ref, bi_ref, bh_ref, out_ref):
  gi = (
      jnp.dot(msg_ref[...], wi_ref[...], preferred_element_type=jnp.float32)
      + bi_ref[...]
  )
  gh = (
      jnp.dot(h_ref[...], wh_ref[...], preferred_element_type=jnp.float32)
      + bh_ref[...]
  )
  H = MEM_DIM
  r = jax.nn.sigmoid(gi[:, :H] + gh[:, :H])
  z = jax.nn.sigmoid(gi[:, H : 2 * H] + gh[:, H : 2 * H])
  n = jnp.tanh(gi[:, 2 * H :] + r * gh[:, 2 * H :])
  out_ref[...] = (1.0 - z) * n + z * h_ref[...]


def _tc_gru(msg, h, wi_t, wh_t, bi, bh):
  grid = (B // _BM,)
  return pl.pallas_call(
      _gru_body,
      grid=grid,
      in_specs=[
          pl.BlockSpec((_BM, MSG_DIM), lambda i: (i, 0)),
          pl.BlockSpec((_BM, MEM_DIM), lambda i: (i, 0)),
          pl.BlockSpec((MSG_DIM, 3 * MEM_DIM), lambda i: (0, 0)),
          pl.BlockSpec((MEM_DIM, 3 * MEM_DIM), lambda i: (0, 0)),
          pl.BlockSpec((1, 3 * MEM_DIM), lambda i: (0, 0)),
          pl.BlockSpec((1, 3 * MEM_DIM), lambda i: (0, 0)),
      ],
      out_specs=pl.BlockSpec((_BM, MEM_DIM), lambda i: (i, 0)),
      out_shape=jax.ShapeDtypeStruct((B, MEM_DIM), jnp.float32),
  )(msg, h, wi_t, wh_t, bi, bh)


@functools.partial(
    pl.kernel,
    mesh=_MESH(),
    out_type=(),
    scratch_types=[
        pltpu.VMEM((NCHUNK, CHUNK), jnp.int32),
        pltpu.VMEM((B_PER_W, MEM_DIM), jnp.float32),
    ]
    + [pltpu.SemaphoreType.DMA] * 5,
)
def _sc_scatter(nids_hbm, hnew_hbm, mem_hbm,
                idx_v, rows_v, s0, s1, s2, s3, ss):
  wid = _worker_id()
  base = wid * B_PER_W
  pltpu.sync_copy(nids_hbm.at[wid], idx_v)
  sems = (s0, s1, s2, s3)
  loads = []
  for k in range(NCHUNK):
    loads.append(
        pltpu.async_copy(
            hnew_hbm.at[pl.ds(base + k * CHUNK, CHUNK)],
            rows_v.at[pl.ds(k * CHUNK, CHUNK)],
            sems[k],
        )
    )
  scatters = []
  for k in range(NCHUNK):
    loads[k].wait()
    scatters.append(
        pltpu.async_copy(
            rows_v.at[pl.ds(k * CHUNK, CHUNK)],
            mem_hbm.at[idx_v.at[k]],
            ss,
        )
    )
  for c in scatters:
    c.wait()


def kernel(unique_nids, unique_msg, time, memory, last_update,
           W_ih, W_hh, b_ih, b_hh):
  nids3 = unique_nids.astype(jnp.int32).reshape(NW, NCHUNK, CHUNK)
  tvals = jnp.full((CHUNK,), time, dtype=jnp.float32)
  lu_ref = jax.new_ref(last_update)
  h = _sc_gather(memory, nids3, tvals, lu_ref)
  h_new = _tc_gru(
      unique_msg, h, W_ih.T, W_hh.T,
      b_ih.reshape(1, -1), b_hh.reshape(1, -1),
  )
  mem_ref = jax.new_ref(memory)
  _sc_scatter(nids3, h_new, mem_ref)
  return mem_ref[...], lu_ref[...]
